# Initial kernel scaffold; baseline (speedup 1.0000x reference)
import jax
import jax.numpy as jnp
from jax.experimental import pallas as pl


def kernel(x, adj):
    raise NotImplementedError("write your pallas kernel here")


# speedup vs baseline: 1.3376x; 1.3376x over previous
import functools

import jax
import jax.numpy as jnp
from jax.experimental import pallas as pl
from jax.experimental.pallas import tpu as pltpu

_MIN_NORM = 1e-15
_EPS_F32 = 4e-3
_N = 4096
_D = 256
_BM = 512


def _artanh(v):
    v = jnp.clip(v, -1.0 + 1e-7, 1.0 - 1e-7)
    return 0.5 * (jnp.log1p(v) - jnp.log1p(-v))


def _hyp_agg_kernel(x_ref, adj_ref, o_ref, xt_ref):
    @pl.when(pl.program_id(0) == 0)
    def _compute_tangent():
        xv = x_ref[...]
        nrm = jnp.maximum(
            jnp.sqrt(jnp.sum(xv * xv, axis=1, keepdims=True)), _MIN_NORM
        )
        scale = _artanh(nrm) / nrm
        xt_ref[...] = (xv * scale).astype(jnp.bfloat16)

    a = adj_ref[...].astype(jnp.bfloat16)
    s = jnp.dot(a, xt_ref[...], preferred_element_type=jnp.float32)
    sn = jnp.maximum(
        jnp.sqrt(jnp.sum(s * s, axis=1, keepdims=True)), _MIN_NORM
    )
    g = jnp.tanh(sn) * (s / sn)
    gn = jnp.maximum(
        jnp.sqrt(jnp.sum(g * g, axis=1, keepdims=True)), _MIN_NORM
    )
    maxnorm = 1.0 - _EPS_F32
    o_ref[...] = jnp.where(gn > maxnorm, g * (maxnorm / gn), g)


@functools.partial(jax.jit, static_argnames=())
def kernel(x, adj):
    return pl.pallas_call(
        _hyp_agg_kernel,
        grid=(_N // _BM,),
        in_specs=[
            pl.BlockSpec((_N, _D), lambda i: (0, 0)),
            pl.BlockSpec((_BM, _N), lambda i: (i, 0)),
        ],
        out_specs=pl.BlockSpec((_BM, _D), lambda i: (i, 0)),
        out_shape=jax.ShapeDtypeStruct((_N, _D), jnp.float32),
        scratch_shapes=[pltpu.VMEM((_N, _D), jnp.bfloat16)],
    )(x, adj)
